# baseline (speedup 1.0000x reference)
import functools

import jax
import jax.numpy as jnp
from jax import lax
from jax.experimental import pallas as pl
from jax.experimental.pallas import tpu as pltpu
from jax.experimental.pallas import tpu_sc as plsc

NUM_EXPERTS = 8
TOP_K = 2
BM = 128
_M = 2048
NBLK = (_M * TOP_K) // BM + NUM_EXPERTS - 1
NR = NBLK * BM
BE_PAD = 128
NW = 32
KC = 8
CH = 16
_CS = 512


def _router_kernel(x_ref, wr_ref, dest_ref, pw_ref, be_ref):
    E = NUM_EXPERTS
    M = x_ref.shape[0]
    logits = lax.dot_general(
        wr_ref[...], x_ref[...], (((1,), (1,)), ((), ())),
        preferred_element_type=jnp.float32)
    probs = jax.nn.sigmoid(logits)
    sub = lax.broadcasted_iota(jnp.int32, (E, M), 0)
    m1 = jnp.max(probs, axis=0, keepdims=True)
    i1 = jnp.min(jnp.where(probs == m1, sub, E), axis=0, keepdims=True)
    masked = jnp.where(sub == i1, -1.0, probs)
    m2 = jnp.max(masked, axis=0, keepdims=True)
    i2 = jnp.min(jnp.where(masked == m2, sub, E), axis=0, keepdims=True)
    denom = m1 + m2
    pw_ref[0:1, :] = m1 / denom
    pw_ref[1:2, :] = m2 / denom

    oh1 = (sub == i1).astype(jnp.float32)
    oh2 = (sub == i2).astype(jnp.float32)
    r_io = lax.broadcasted_iota(jnp.int32, (_CS, _CS), 0)
    c_io = lax.broadcasted_iota(jnp.int32, (_CS, _CS), 1)
    upper = (r_io < c_io).astype(jnp.float32)

    def excl_cumsum(oh, tot):
        parts = []
        for c in range(M // _CS):
            blk = oh[:, c * _CS:(c + 1) * _CS]
            rc = lax.dot_general(blk, upper, (((1,), (0,)), ((), ())),
                                 preferred_element_type=jnp.float32,
                                 precision=lax.Precision.HIGHEST) + tot
            parts.append(rc)
            tot = tot + jnp.sum(blk, axis=1, keepdims=True)
        return jnp.concatenate(parts, axis=1), tot

    zero = jnp.zeros((E, 1), jnp.float32)
    r1, tot1 = excl_cumsum(oh1, zero)
    r2, counts = excl_cumsum(oh2, tot1)
    padded = jnp.floor((counts + (BM - 1)) / BM) * BM
    er_io = lax.broadcasted_iota(jnp.int32, (E, E), 0)
    ec_io = lax.broadcasted_iota(jnp.int32, (E, E), 1)
    lower = (ec_io < er_io).astype(jnp.float32)
    offs = lax.dot_general(lower, padded, (((1,), (0,)), ((), ())),
                           preferred_element_type=jnp.float32,
                           precision=lax.Precision.HIGHEST)
    dest1 = jnp.sum(oh1 * (offs + r1), axis=0, keepdims=True)
    dest2 = jnp.sum(oh2 * (offs + r2), axis=0, keepdims=True)
    dest_ref[0:1, :] = dest1.astype(jnp.int32)
    dest_ref[1:2, :] = dest2.astype(jnp.int32)

    b_io = lax.broadcasted_iota(jnp.int32, (1, BE_PAD), 1).astype(jnp.float32)
    end_blk = (offs + padded) / BM
    esel = lax.broadcasted_iota(jnp.int32, (E, 1), 0)
    be = jnp.zeros((1, BE_PAD), jnp.float32)
    for e in range(E):
        eb_e = jnp.sum(jnp.where(esel == e, end_blk, 0.0), axis=0, keepdims=True)
        be = be + (b_io >= eb_e).astype(jnp.float32)
    be_ref[...] = jnp.minimum(be, E - 1).astype(jnp.int32)


def _gateup_kernel(be_sref, xs_ref, wg_ref, wu_ref, h_ref):
    x = xs_ref[...].astype(jnp.bfloat16)
    wg = wg_ref[0].astype(jnp.bfloat16)
    wu = wu_ref[0].astype(jnp.bfloat16)
    g = jnp.dot(x, wg.T, preferred_element_type=jnp.float32)
    u = jnp.dot(x, wu.T, preferred_element_type=jnp.float32)
    h_ref[...] = ((g * jax.nn.sigmoid(g)) * u).astype(jnp.bfloat16)


def _down_kernel(be_sref, h_ref, wd_ref, ys_ref):
    wd = wd_ref[0].astype(jnp.bfloat16)
    ys_ref[...] = jnp.dot(h_ref[...], wd.T, preferred_element_type=jnp.float32)


def _combine_kernel(g1_ref, g2_ref, pwt_ref, out_ref):
    pwt = pwt_ref[...]
    out_ref[...] = pwt[:, 0:1] * g1_ref[...] + pwt[:, 1:2] * g2_ref[...]


def _make_scatter(H, dtype):
    @functools.partial(
        pl.kernel,
        mesh=plsc.VectorSubcoreMesh(core_axis_name="c", subcore_axis_name="s"),
        out_type=jax.ShapeDtypeStruct((NR, H), dtype),
        scratch_types=[
            pltpu.VMEM((KC, CH), jnp.int32),
            pltpu.VMEM((CH, H), dtype),
            pltpu.VMEM((CH, H), dtype),
            pltpu.SemaphoreType.DMA,
            pltpu.SemaphoreType.DMA,
            pltpu.SemaphoreType.DMA,
            pltpu.SemaphoreType.DMA,
        ],
    )
    def scatter_k(x_hbm, idx_hbm, xs_hbm, idx_v, buf0, buf1,
                  si0, si1, so0, so1):
        wid = lax.axis_index("s") * 2 + lax.axis_index("c")
        t0 = (wid % 16) * (KC * CH)
        pltpu.sync_copy(idx_hbm.at[wid], idx_v)
        bufs = (buf0, buf1)
        sis = (si0, si1)
        sos = (so0, so1)
        outs = [None] * KC
        for j in range(KC):
            if j >= 2:
                outs[j - 2].wait()
            pltpu.async_copy(x_hbm.at[pl.ds(t0 + j * CH, CH)], bufs[j % 2],
                             sis[j % 2]).wait()
            outs[j] = pltpu.async_copy(bufs[j % 2], xs_hbm.at[idx_v.at[j]],
                                       sos[j % 2])
        outs[KC - 2].wait()
        outs[KC - 1].wait()

    return scatter_k


def _make_gather(H, dtype):
    @functools.partial(
        pl.kernel,
        mesh=plsc.VectorSubcoreMesh(core_axis_name="c", subcore_axis_name="s"),
        out_type=jax.ShapeDtypeStruct((NW * KC * CH, H), dtype),
        scratch_types=[
            pltpu.VMEM((KC, CH), jnp.int32),
            pltpu.VMEM((CH, H), dtype),
            pltpu.VMEM((CH, H), dtype),
            pltpu.SemaphoreType.DMA,
            pltpu.SemaphoreType.DMA,
            pltpu.SemaphoreType.DMA,
            pltpu.SemaphoreType.DMA,
        ],
    )
    def gather_k(ys_hbm, idx_hbm, g_hbm, idx_v, buf0, buf1,
                 si0, si1, so0, so1):
        wid = lax.axis_index("s") * 2 + lax.axis_index("c")
        p0 = wid * (KC * CH)
        pltpu.sync_copy(idx_hbm.at[wid], idx_v)
        bufs = (buf0, buf1)
        sis = (si0, si1)
        sos = (so0, so1)
        outs = [None] * KC
        for j in range(KC):
            if j >= 2:
                outs[j - 2].wait()
            pltpu.async_copy(ys_hbm.at[idx_v.at[j]], bufs[j % 2],
                             sis[j % 2]).wait()
            outs[j] = pltpu.async_copy(bufs[j % 2],
                                       g_hbm.at[pl.ds(p0 + j * CH, CH)],
                                       sos[j % 2])
        outs[KC - 2].wait()
        outs[KC - 1].wait()

    return gather_k


def kernel(hidden_states, w_router, w_gate, w_up, w_down):
    M, H = hidden_states.shape
    E, I, _ = w_gate.shape

    dest, pw, be = pl.pallas_call(
        _router_kernel,
        out_shape=(
            jax.ShapeDtypeStruct((TOP_K, M), jnp.int32),
            jax.ShapeDtypeStruct((TOP_K, M), jnp.float32),
            jax.ShapeDtypeStruct((1, BE_PAD), jnp.int32),
        ),
    )(hidden_states, w_router)

    idx3 = dest.reshape(NW, KC, CH)
    xs = _make_scatter(H, jnp.float32)(hidden_states, idx3)

    be_flat = be.reshape(BE_PAD)
    h = pl.pallas_call(
        _gateup_kernel,
        grid_spec=pltpu.PrefetchScalarGridSpec(
            num_scalar_prefetch=1,
            grid=(NBLK,),
            in_specs=[
                pl.BlockSpec((BM, H), lambda b, be_ref: (b, 0)),
                pl.BlockSpec((1, I, H), lambda b, be_ref: (be_ref[b], 0, 0)),
                pl.BlockSpec((1, I, H), lambda b, be_ref: (be_ref[b], 0, 0)),
            ],
            out_specs=pl.BlockSpec((BM, I), lambda b, be_ref: (b, 0)),
        ),
        out_shape=jax.ShapeDtypeStruct((NR, I), jnp.bfloat16),
    )(be_flat, xs, w_gate, w_up)
    ys = pl.pallas_call(
        _down_kernel,
        grid_spec=pltpu.PrefetchScalarGridSpec(
            num_scalar_prefetch=1,
            grid=(NBLK,),
            in_specs=[
                pl.BlockSpec((BM, I), lambda b, be_ref: (b, 0)),
                pl.BlockSpec((1, H, I), lambda b, be_ref: (be_ref[b], 0, 0)),
            ],
            out_specs=pl.BlockSpec((BM, H), lambda b, be_ref: (b, 0)),
        ),
        out_shape=jax.ShapeDtypeStruct((NR, H), jnp.float32),
    )(be_flat, h, w_down)

    g = _make_gather(H, jnp.float32)(ys, idx3)

    BT = 512
    out = pl.pallas_call(
        _combine_kernel,
        grid=(M // BT,),
        in_specs=[
            pl.BlockSpec((BT, H), lambda t: (t, 0)),
            pl.BlockSpec((BT, H), lambda t: (t + M // BT, 0)),
            pl.BlockSpec((BT, TOP_K), lambda t: (t, 0)),
        ],
        out_specs=pl.BlockSpec((BT, H), lambda t: (t, 0)),
        out_shape=jax.ShapeDtypeStruct((M, H), jnp.float32),
    )(g, g, pw.T)
    return out


# speedup vs baseline: 1.3516x; 1.3516x over previous
import functools

import jax
import jax.numpy as jnp
from jax import lax
from jax.experimental import pallas as pl
from jax.experimental.pallas import tpu as pltpu
from jax.experimental.pallas import tpu_sc as plsc

NUM_EXPERTS = 8
TOP_K = 2
BM = 256
_M = 2048
NBLK = (_M * TOP_K) // BM + NUM_EXPERTS - 1
NR = NBLK * BM
BE_PAD = 128
NW = 32
KC = 8
CH = 16
_CS = 512


def _router_kernel(x_ref, wr_ref, dest_ref, pw_ref, be_ref):
    E = NUM_EXPERTS
    M = x_ref.shape[0]
    logits = lax.dot_general(
        wr_ref[...], x_ref[...], (((1,), (1,)), ((), ())),
        preferred_element_type=jnp.float32)
    probs = jax.nn.sigmoid(logits)
    sub = lax.broadcasted_iota(jnp.int32, (E, M), 0)
    m1 = jnp.max(probs, axis=0, keepdims=True)
    i1 = jnp.min(jnp.where(probs == m1, sub, E), axis=0, keepdims=True)
    masked = jnp.where(sub == i1, -1.0, probs)
    m2 = jnp.max(masked, axis=0, keepdims=True)
    i2 = jnp.min(jnp.where(masked == m2, sub, E), axis=0, keepdims=True)
    denom = m1 + m2
    pw_ref[0:1, :] = m1 / denom
    pw_ref[1:2, :] = m2 / denom

    oh1 = (sub == i1).astype(jnp.float32)
    oh2 = (sub == i2).astype(jnp.float32)
    r_io = lax.broadcasted_iota(jnp.int32, (_CS, _CS), 0)
    c_io = lax.broadcasted_iota(jnp.int32, (_CS, _CS), 1)
    upper = (r_io < c_io).astype(jnp.float32)

    def excl_cumsum(oh, tot):
        parts = []
        for c in range(M // _CS):
            blk = oh[:, c * _CS:(c + 1) * _CS]
            rc = lax.dot_general(blk, upper, (((1,), (0,)), ((), ())),
                                 preferred_element_type=jnp.float32,
                                 precision=lax.Precision.HIGHEST) + tot
            parts.append(rc)
            tot = tot + jnp.sum(blk, axis=1, keepdims=True)
        return jnp.concatenate(parts, axis=1), tot

    zero = jnp.zeros((E, 1), jnp.float32)
    r1, tot1 = excl_cumsum(oh1, zero)
    r2, counts = excl_cumsum(oh2, tot1)
    padded = jnp.floor((counts + (BM - 1)) / BM) * BM
    er_io = lax.broadcasted_iota(jnp.int32, (E, E), 0)
    ec_io = lax.broadcasted_iota(jnp.int32, (E, E), 1)
    lower = (ec_io < er_io).astype(jnp.float32)
    offs = lax.dot_general(lower, padded, (((1,), (0,)), ((), ())),
                           preferred_element_type=jnp.float32,
                           precision=lax.Precision.HIGHEST)
    dest1 = jnp.sum(oh1 * (offs + r1), axis=0, keepdims=True)
    dest2 = jnp.sum(oh2 * (offs + r2), axis=0, keepdims=True)
    dest_ref[0:1, :] = dest1.astype(jnp.int32)
    dest_ref[1:2, :] = dest2.astype(jnp.int32)

    b_io = lax.broadcasted_iota(jnp.int32, (1, BE_PAD), 1).astype(jnp.float32)
    end_blk = (offs + padded) / BM
    esel = lax.broadcasted_iota(jnp.int32, (E, 1), 0)
    be = jnp.zeros((1, BE_PAD), jnp.float32)
    for e in range(E):
        eb_e = jnp.sum(jnp.where(esel == e, end_blk, 0.0), axis=0, keepdims=True)
        be = be + (b_io >= eb_e).astype(jnp.float32)
    be_ref[...] = jnp.minimum(be, E - 1).astype(jnp.int32)


def _gateup_kernel(be_sref, xs_ref, wg_ref, wu_ref, h_ref):
    x = xs_ref[...].astype(jnp.bfloat16)
    wg = wg_ref[0].astype(jnp.bfloat16)
    wu = wu_ref[0].astype(jnp.bfloat16)
    g = jnp.dot(x, wg.T, preferred_element_type=jnp.float32)
    u = jnp.dot(x, wu.T, preferred_element_type=jnp.float32)
    h_ref[...] = ((g * jax.nn.sigmoid(g)) * u).astype(jnp.bfloat16)


def _down_kernel(be_sref, h_ref, wd_ref, ys_ref):
    wd = wd_ref[0].astype(jnp.bfloat16)
    ys_ref[...] = jnp.dot(h_ref[...], wd.T, preferred_element_type=jnp.float32)


def _combine_kernel(g1_ref, g2_ref, pwt_ref, out_ref):
    pwt = pwt_ref[...]
    out_ref[...] = pwt[:, 0:1] * g1_ref[...] + pwt[:, 1:2] * g2_ref[...]


def _make_scatter(H, dtype):
    @functools.partial(
        pl.kernel,
        mesh=plsc.VectorSubcoreMesh(core_axis_name="c", subcore_axis_name="s"),
        out_type=jax.ShapeDtypeStruct((NR, H), dtype),
        scratch_types=[
            pltpu.VMEM((KC, CH), jnp.int32),
            pltpu.VMEM((CH, H), dtype),
            pltpu.VMEM((CH, H), dtype),
            pltpu.SemaphoreType.DMA,
            pltpu.SemaphoreType.DMA,
            pltpu.SemaphoreType.DMA,
            pltpu.SemaphoreType.DMA,
        ],
    )
    def scatter_k(x_hbm, idx_hbm, xs_hbm, idx_v, buf0, buf1,
                  si0, si1, so0, so1):
        wid = lax.axis_index("s") * 2 + lax.axis_index("c")
        t0 = wid * (KC // 2 * CH)
        pltpu.sync_copy(idx_hbm.at[wid], idx_v)
        bufs = (buf0, buf1)
        sis = (si0, si1)
        sos = (so0, so1)
        outs = [None] * KC
        nch = KC // 2
        for c in range(nch):
            if c >= 2:
                outs[2 * (c - 2)].wait()
                outs[2 * (c - 2) + 1].wait()
            pltpu.async_copy(x_hbm.at[pl.ds(t0 + c * CH, CH)], bufs[c % 2],
                             sis[c % 2]).wait()
            outs[2 * c] = pltpu.async_copy(
                bufs[c % 2], xs_hbm.at[idx_v.at[c]], sos[c % 2])
            outs[2 * c + 1] = pltpu.async_copy(
                bufs[c % 2], xs_hbm.at[idx_v.at[nch + c]], sos[c % 2])
        for c in range(max(0, nch - 2), nch):
            outs[2 * c].wait()
            outs[2 * c + 1].wait()

    return scatter_k


def _make_gather(H, dtype):
    @functools.partial(
        pl.kernel,
        mesh=plsc.VectorSubcoreMesh(core_axis_name="c", subcore_axis_name="s"),
        out_type=jax.ShapeDtypeStruct((NW * KC * CH, H), dtype),
        scratch_types=[
            pltpu.VMEM((KC, CH), jnp.int32),
            pltpu.VMEM((CH, H), dtype),
            pltpu.VMEM((CH, H), dtype),
            pltpu.SemaphoreType.DMA,
            pltpu.SemaphoreType.DMA,
            pltpu.SemaphoreType.DMA,
            pltpu.SemaphoreType.DMA,
        ],
    )
    def gather_k(ys_hbm, idx_hbm, g_hbm, idx_v, buf0, buf1,
                 si0, si1, so0, so1):
        wid = lax.axis_index("s") * 2 + lax.axis_index("c")
        t0 = wid * (KC // 2 * CH)
        nch = KC // 2
        half = NW * (KC // 2) * CH
        pltpu.sync_copy(idx_hbm.at[wid], idx_v)
        bufs = (buf0, buf1)
        sis = (si0, si1)
        sos = (so0, so1)
        outs = [None] * KC
        for j in range(KC):
            base = (t0 + j * CH) if j < nch else (half + t0 + (j - nch) * CH)
            if j >= 2:
                outs[j - 2].wait()
            pltpu.async_copy(ys_hbm.at[idx_v.at[j]], bufs[j % 2],
                             sis[j % 2]).wait()
            outs[j] = pltpu.async_copy(bufs[j % 2],
                                       g_hbm.at[pl.ds(base, CH)],
                                       sos[j % 2])
        outs[KC - 2].wait()
        outs[KC - 1].wait()

    return gather_k


def kernel(hidden_states, w_router, w_gate, w_up, w_down):
    M, H = hidden_states.shape
    E, I, _ = w_gate.shape

    dest, pw, be = pl.pallas_call(
        _router_kernel,
        out_shape=(
            jax.ShapeDtypeStruct((TOP_K, M), jnp.int32),
            jax.ShapeDtypeStruct((TOP_K, M), jnp.float32),
            jax.ShapeDtypeStruct((1, BE_PAD), jnp.int32),
        ),
    )(hidden_states, w_router)

    idx3 = (dest.reshape(TOP_K, NW, KC // TOP_K, CH)
            .transpose(1, 0, 2, 3).reshape(NW, KC, CH))
    xs = _make_scatter(H, jnp.float32)(hidden_states, idx3)

    be_flat = be.reshape(BE_PAD)
    h = pl.pallas_call(
        _gateup_kernel,
        grid_spec=pltpu.PrefetchScalarGridSpec(
            num_scalar_prefetch=1,
            grid=(NBLK,),
            in_specs=[
                pl.BlockSpec((BM, H), lambda b, be_ref: (b, 0)),
                pl.BlockSpec((1, I, H), lambda b, be_ref: (be_ref[b], 0, 0)),
                pl.BlockSpec((1, I, H), lambda b, be_ref: (be_ref[b], 0, 0)),
            ],
            out_specs=pl.BlockSpec((BM, I), lambda b, be_ref: (b, 0)),
        ),
        out_shape=jax.ShapeDtypeStruct((NR, I), jnp.bfloat16),
    )(be_flat, xs, w_gate, w_up)
    ys = pl.pallas_call(
        _down_kernel,
        grid_spec=pltpu.PrefetchScalarGridSpec(
            num_scalar_prefetch=1,
            grid=(NBLK,),
            in_specs=[
                pl.BlockSpec((BM, I), lambda b, be_ref: (b, 0)),
                pl.BlockSpec((1, H, I), lambda b, be_ref: (be_ref[b], 0, 0)),
            ],
            out_specs=pl.BlockSpec((BM, H), lambda b, be_ref: (b, 0)),
        ),
        out_shape=jax.ShapeDtypeStruct((NR, H), jnp.float32),
    )(be_flat, h, w_down)

    g = _make_gather(H, jnp.float32)(ys, idx3)

    BT = 512
    out = pl.pallas_call(
        _combine_kernel,
        grid=(M // BT,),
        in_specs=[
            pl.BlockSpec((BT, H), lambda t: (t, 0)),
            pl.BlockSpec((BT, H), lambda t: (t + M // BT, 0)),
            pl.BlockSpec((BT, TOP_K), lambda t: (t, 0)),
        ],
        out_specs=pl.BlockSpec((BT, H), lambda t: (t, 0)),
        out_shape=jax.ShapeDtypeStruct((M, H), jnp.float32),
    )(g, g, pw.T)
    return out


# speedup vs baseline: 1.3680x; 1.0122x over previous
import functools

import jax
import jax.numpy as jnp
from jax import lax
from jax.experimental import pallas as pl
from jax.experimental.pallas import tpu as pltpu
from jax.experimental.pallas import tpu_sc as plsc

NUM_EXPERTS = 8
TOP_K = 2
BM = 256
_M = 2048
NBLK = (_M * TOP_K) // BM + NUM_EXPERTS - 1
NR = NBLK * BM
BE_PAD = 128
NW = 32
KC = 8
CH = 16
_CS = 512


def _router_kernel(x_ref, wr_ref, dest_ref, pw_ref, be_ref):
    E = NUM_EXPERTS
    M = x_ref.shape[0]
    logits = lax.dot_general(
        wr_ref[...], x_ref[...], (((1,), (1,)), ((), ())),
        preferred_element_type=jnp.float32)
    probs = jax.nn.sigmoid(logits)
    sub = lax.broadcasted_iota(jnp.int32, (E, M), 0)
    m1 = jnp.max(probs, axis=0, keepdims=True)
    i1 = jnp.min(jnp.where(probs == m1, sub, E), axis=0, keepdims=True)
    masked = jnp.where(sub == i1, -1.0, probs)
    m2 = jnp.max(masked, axis=0, keepdims=True)
    i2 = jnp.min(jnp.where(masked == m2, sub, E), axis=0, keepdims=True)
    denom = m1 + m2
    pw_ref[0:1, :] = m1 / denom
    pw_ref[1:2, :] = m2 / denom

    oh1 = (sub == i1).astype(jnp.float32)
    oh2 = (sub == i2).astype(jnp.float32)
    r_io = lax.broadcasted_iota(jnp.int32, (_CS, _CS), 0)
    c_io = lax.broadcasted_iota(jnp.int32, (_CS, _CS), 1)
    upper = (r_io < c_io).astype(jnp.float32)

    def excl_cumsum(oh, tot):
        parts = []
        for c in range(M // _CS):
            blk = oh[:, c * _CS:(c + 1) * _CS]
            rc = lax.dot_general(blk, upper, (((1,), (0,)), ((), ())),
                                 preferred_element_type=jnp.float32) + tot
            parts.append(rc)
            tot = tot + jnp.sum(blk, axis=1, keepdims=True)
        return jnp.concatenate(parts, axis=1), tot

    zero = jnp.zeros((E, 1), jnp.float32)
    r1, tot1 = excl_cumsum(oh1, zero)
    r2, counts = excl_cumsum(oh2, tot1)
    padded = jnp.floor((counts + (BM - 1)) / BM) * BM
    er_io = lax.broadcasted_iota(jnp.int32, (E, E), 0)
    ec_io = lax.broadcasted_iota(jnp.int32, (E, E), 1)
    lower = (ec_io < er_io).astype(jnp.float32)
    offs = lax.dot_general(lower, padded, (((1,), (0,)), ((), ())),
                           preferred_element_type=jnp.float32,
                           precision=lax.Precision.HIGHEST)
    dest1 = jnp.sum(oh1 * (offs + r1), axis=0, keepdims=True)
    dest2 = jnp.sum(oh2 * (offs + r2), axis=0, keepdims=True)
    dest_ref[0:1, :] = dest1.astype(jnp.int32)
    dest_ref[1:2, :] = dest2.astype(jnp.int32)

    b_io = lax.broadcasted_iota(jnp.int32, (1, BE_PAD), 1).astype(jnp.float32)
    end_blk = (offs + padded) / BM
    esel = lax.broadcasted_iota(jnp.int32, (E, 1), 0)
    be = jnp.zeros((1, BE_PAD), jnp.float32)
    for e in range(E):
        eb_e = jnp.sum(jnp.where(esel == e, end_blk, 0.0), axis=0, keepdims=True)
        be = be + (b_io >= eb_e).astype(jnp.float32)
    be_ref[...] = jnp.minimum(be, E - 1).astype(jnp.int32)


def _gateup_kernel(be_sref, xs_ref, wg_ref, wu_ref, h_ref):
    x = xs_ref[...].astype(jnp.bfloat16)
    wg = wg_ref[0].astype(jnp.bfloat16)
    wu = wu_ref[0].astype(jnp.bfloat16)
    g = jnp.dot(x, wg.T, preferred_element_type=jnp.float32)
    u = jnp.dot(x, wu.T, preferred_element_type=jnp.float32)
    h_ref[...] = ((g * jax.nn.sigmoid(g)) * u).astype(jnp.bfloat16)


def _down_kernel(be_sref, h_ref, wd_ref, ys_ref):
    wd = wd_ref[0].astype(jnp.bfloat16)
    ys_ref[...] = jnp.dot(h_ref[...], wd.T, preferred_element_type=jnp.float32)


def _combine_kernel(g1_ref, g2_ref, pwt_ref, out_ref):
    pwt = pwt_ref[...]
    out_ref[...] = pwt[:, 0:1] * g1_ref[...] + pwt[:, 1:2] * g2_ref[...]


def _make_scatter(H, dtype):
    @functools.partial(
        pl.kernel,
        mesh=plsc.VectorSubcoreMesh(core_axis_name="c", subcore_axis_name="s"),
        out_type=jax.ShapeDtypeStruct((NR, H), dtype),
        scratch_types=[
            pltpu.VMEM((KC, CH), jnp.int32),
            pltpu.VMEM((CH, H), dtype),
            pltpu.VMEM((CH, H), dtype),
            pltpu.SemaphoreType.DMA,
            pltpu.SemaphoreType.DMA,
            pltpu.SemaphoreType.DMA,
            pltpu.SemaphoreType.DMA,
        ],
    )
    def scatter_k(x_hbm, idx_hbm, xs_hbm, idx_v, buf0, buf1,
                  si0, si1, so0, so1):
        wid = lax.axis_index("s") * 2 + lax.axis_index("c")
        t0 = wid * (KC // 2 * CH)
        pltpu.sync_copy(idx_hbm.at[wid], idx_v)
        bufs = (buf0, buf1)
        sis = (si0, si1)
        sos = (so0, so1)
        outs = [None] * KC
        nch = KC // 2
        for c in range(nch):
            if c >= 2:
                outs[2 * (c - 2)].wait()
                outs[2 * (c - 2) + 1].wait()
            pltpu.async_copy(x_hbm.at[pl.ds(t0 + c * CH, CH)], bufs[c % 2],
                             sis[c % 2]).wait()
            outs[2 * c] = pltpu.async_copy(
                bufs[c % 2], xs_hbm.at[idx_v.at[c]], sos[c % 2])
            outs[2 * c + 1] = pltpu.async_copy(
                bufs[c % 2], xs_hbm.at[idx_v.at[nch + c]], sos[c % 2])
        for c in range(max(0, nch - 2), nch):
            outs[2 * c].wait()
            outs[2 * c + 1].wait()

    return scatter_k


def _make_gather(H, dtype):
    @functools.partial(
        pl.kernel,
        mesh=plsc.VectorSubcoreMesh(core_axis_name="c", subcore_axis_name="s"),
        out_type=jax.ShapeDtypeStruct((NW * KC * CH, H), dtype),
        scratch_types=[
            pltpu.VMEM((KC, CH), jnp.int32),
            pltpu.VMEM((CH, H), dtype),
            pltpu.VMEM((CH, H), dtype),
            pltpu.SemaphoreType.DMA,
            pltpu.SemaphoreType.DMA,
            pltpu.SemaphoreType.DMA,
            pltpu.SemaphoreType.DMA,
        ],
    )
    def gather_k(ys_hbm, idx_hbm, g_hbm, idx_v, buf0, buf1,
                 si0, si1, so0, so1):
        wid = lax.axis_index("s") * 2 + lax.axis_index("c")
        t0 = wid * (KC // 2 * CH)
        nch = KC // 2
        half = NW * (KC // 2) * CH
        pltpu.sync_copy(idx_hbm.at[wid], idx_v)
        bufs = (buf0, buf1)
        sis = (si0, si1)
        sos = (so0, so1)
        outs = [None] * KC
        for j in range(KC):
            base = (t0 + j * CH) if j < nch else (half + t0 + (j - nch) * CH)
            if j >= 2:
                outs[j - 2].wait()
            pltpu.async_copy(ys_hbm.at[idx_v.at[j]], bufs[j % 2],
                             sis[j % 2]).wait()
            outs[j] = pltpu.async_copy(bufs[j % 2],
                                       g_hbm.at[pl.ds(base, CH)],
                                       sos[j % 2])
        outs[KC - 2].wait()
        outs[KC - 1].wait()

    return gather_k


def kernel(hidden_states, w_router, w_gate, w_up, w_down):
    M, H = hidden_states.shape
    E, I, _ = w_gate.shape

    dest, pw, be = pl.pallas_call(
        _router_kernel,
        out_shape=(
            jax.ShapeDtypeStruct((TOP_K, M), jnp.int32),
            jax.ShapeDtypeStruct((TOP_K, M), jnp.float32),
            jax.ShapeDtypeStruct((1, BE_PAD), jnp.int32),
        ),
    )(hidden_states, w_router)

    idx3 = (dest.reshape(TOP_K, NW, KC // TOP_K, CH)
            .transpose(1, 0, 2, 3).reshape(NW, KC, CH))
    xs = _make_scatter(H, jnp.float32)(hidden_states, idx3)

    be_flat = be.reshape(BE_PAD)
    h = pl.pallas_call(
        _gateup_kernel,
        grid_spec=pltpu.PrefetchScalarGridSpec(
            num_scalar_prefetch=1,
            grid=(NBLK,),
            in_specs=[
                pl.BlockSpec((BM, H), lambda b, be_ref: (b, 0)),
                pl.BlockSpec((1, I, H), lambda b, be_ref: (be_ref[b], 0, 0)),
                pl.BlockSpec((1, I, H), lambda b, be_ref: (be_ref[b], 0, 0)),
            ],
            out_specs=pl.BlockSpec((BM, I), lambda b, be_ref: (b, 0)),
        ),
        out_shape=jax.ShapeDtypeStruct((NR, I), jnp.bfloat16),
    )(be_flat, xs, w_gate, w_up)
    ys = pl.pallas_call(
        _down_kernel,
        grid_spec=pltpu.PrefetchScalarGridSpec(
            num_scalar_prefetch=1,
            grid=(NBLK,),
            in_specs=[
                pl.BlockSpec((BM, I), lambda b, be_ref: (b, 0)),
                pl.BlockSpec((1, H, I), lambda b, be_ref: (be_ref[b], 0, 0)),
            ],
            out_specs=pl.BlockSpec((BM, H), lambda b, be_ref: (b, 0)),
        ),
        out_shape=jax.ShapeDtypeStruct((NR, H), jnp.float32),
    )(be_flat, h, w_down)

    g = _make_gather(H, jnp.float32)(ys, idx3)

    BT = 512
    out = pl.pallas_call(
        _combine_kernel,
        grid=(M // BT,),
        in_specs=[
            pl.BlockSpec((BT, H), lambda t: (t, 0)),
            pl.BlockSpec((BT, H), lambda t: (t + M // BT, 0)),
            pl.BlockSpec((BT, TOP_K), lambda t: (t, 0)),
        ],
        out_specs=pl.BlockSpec((BT, H), lambda t: (t, 0)),
        out_shape=jax.ShapeDtypeStruct((M, H), jnp.float32),
    )(g, g, pw.T)
    return out
